# baseline (speedup 1.0000x reference)
import functools

import jax
import jax.numpy as jnp
from jax import lax
from jax.experimental import pallas as pl
from jax.experimental.pallas import tpu as pltpu
from jax.experimental.pallas import tpu_sc as plsc

_N0, _N1, _N2, _NNET = 10000, 2500, 625, 8000
_D = 128
_NC, _NS = 2, 16
_NW = _NC * _NS
_K = 128
_BLK = 2048
_BM = 512


def _cdiv(a, b):
    return -(-a // b)


def _pad_edges(src, dst, step, dst_pad):
    e = src.shape[0]
    e_pad = _cdiv(e, step) * step
    if e_pad != e:
        src = jnp.concatenate([src, jnp.zeros((e_pad - e,), jnp.int32)])
        dst = jnp.concatenate([dst, jnp.full((e_pad - e,), dst_pad, jnp.int32)])
    return src, dst


def _n_acc(n_out):
    return _cdiv(n_out + 1, _NS * _K) * _NS * _K


_SPLIT = 8064


@functools.lru_cache(maxsize=None)
def _make_seg_sum(e_pad, n_src, n_out):
    n_acc = _n_acc(n_out)
    dual = n_acc > 8192
    if dual:
        ra, rb = 8192, n_acc - 8192
        dummy_a, dummy_b = _SPLIT, rb - 1
    else:
        ra, rb = n_acc, 0
    e_core = e_pad // _NC
    e_tile = e_core // _NS
    nt = e_tile // _K
    mesh = plsc.VectorSubcoreMesh(core_axis_name="c", subcore_axis_name="s")

    def body(feat, srcs, dsts, rowidx, out_sum, *refs):
        if dual:
            (src_v, dst_v, idx_v, rows_v, acc_a, acc_b, sem) = refs
            accs = ((acc_a, ra), (acc_b, rb))
        else:
            (src_v, dst_v, idx_v, rows_v, acc_a, sem) = refs
            accs = ((acc_a, ra),)
        c = lax.axis_index("c")
        s = lax.axis_index("s")

        def fill(i, _):
            for h in range(_D // 16):
                rows_v[i, pl.ds(16 * h, 16)] = jnp.zeros((16,), jnp.float32)
            return 0
        lax.fori_loop(0, _K, fill, 0, unroll=False)

        for acc, nr in accs:
            rps = nr // _NS
            for z in range(rps // _K):
                base = s * rps + z * _K
                pltpu.sync_copy(rowidx.at[pl.ds(base, _K)], idx_v)
                pltpu.sync_copy(rows_v, acc.at[idx_v])
        plsc.subcore_barrier()

        ebase = c * e_core + s * e_tile

        def ebody(t, _):
            o = ebase + t * _K
            pltpu.sync_copy(srcs.at[pl.ds(o, _K)], src_v)
            pltpu.sync_copy(dsts.at[pl.ds(o, _K)], dst_v)
            pltpu.async_copy(feat.at[src_v], rows_v, sem).wait()
            if dual:
                def route(h, _):
                    d16 = dst_v[pl.ds(16 * h, 16)]
                    ia = jnp.where(d16 < _SPLIT, d16, jnp.int32(dummy_a))
                    ib = jnp.where(d16 >= _SPLIT, d16 - _SPLIT,
                                   jnp.int32(dummy_b))
                    idx_v[pl.ds(16 * h, 16)] = ia
                    dst_v[pl.ds(16 * h, 16)] = ib
                    return 0
                lax.fori_loop(0, _K // 16, route, 0, unroll=False)
                pltpu.sync_copy(rows_v, acc_a.at[idx_v], add=True)
                pltpu.sync_copy(rows_v, acc_b.at[dst_v], add=True)
            else:
                pltpu.sync_copy(rows_v, acc_a.at[dst_v], add=True)
            return 0
        lax.fori_loop(0, nt, ebody, 0, unroll=False)
        plsc.subcore_barrier()

        rbase = 0
        for acc, nr in accs:
            rps = nr // _NS
            for z in range(rps // _K):
                base = s * rps + z * _K
                pltpu.sync_copy(rowidx.at[pl.ds(base, _K)], idx_v)
                pltpu.sync_copy(acc.at[idx_v], rows_v)
                pltpu.sync_copy(rows_v,
                                out_sum.at[pl.ds(c * n_acc + rbase + base, _K)])
            rbase += nr

    scratch = [
        pltpu.VMEM((_K,), jnp.int32),
        pltpu.VMEM((_K,), jnp.int32),
        pltpu.VMEM((_K,), jnp.int32),
        pltpu.VMEM((_K, _D), jnp.float32),
        pltpu.VMEM_SHARED((ra, _D), jnp.float32),
    ]
    if dual:
        scratch.append(pltpu.VMEM_SHARED((rb, _D), jnp.float32))
    scratch.append(pltpu.SemaphoreType.DMA)
    return pl.kernel(
        body,
        out_type=jax.ShapeDtypeStruct((_NC * n_acc, _D), jnp.float32),
        mesh=mesh,
        scratch_types=scratch,
    ), n_acc, dual


_ROWIDX_LEN = 32768


def _seg_sum(feat, src, dst, n_out):
    step = _NW * _K
    src, dst = _pad_edges(src, dst, step, n_out)
    k, n_acc, dual = _make_seg_sum(src.shape[0], feat.shape[0], n_out)
    rowidx = jnp.arange(_ROWIDX_LEN, dtype=jnp.int32)
    s = k(feat, src, dst, rowidx).reshape(_NC, n_acc, _D)
    if dual:
        s = jnp.concatenate([s[:, :_SPLIT], s[:, 8192:8192 + n_out - _SPLIT]],
                            axis=1)
    else:
        s = s[:, :n_out]
    return s


def _deg_all(dst_lists):
    ones_t = jnp.ones((8, _D), jnp.float32)
    outs = []
    for dst, n_out in dst_lists:
        s = _seg_sum(ones_t, jnp.zeros_like(dst), dst, n_out)
        outs.append(s[:, :, :16])
    return outs


def _lane_gather(v, idx):
    return lax.gather(
        v, idx[:, None],
        lax.GatherDimensionNumbers(offset_dims=(), collapsed_slice_dims=(0,),
                                   start_index_map=(0,)),
        (1,), mode=lax.GatherScatterMode.PROMISE_IN_BOUNDS)


def _prefix_sum16(v):
    lane = lax.iota(jnp.int32, 16)
    for k in (1, 2, 4, 8):
        sh = _lane_gather(v, jnp.where(lane >= k, lane - k, 0))
        v = v + jnp.where(lane >= k, sh, jnp.int32(0))
    return v


@functools.lru_cache(maxsize=None)
def _make_seg_max(e_pad, n_src):
    rows_per = _NNET // _NW
    nblk = e_pad // _BLK
    mesh = plsc.VectorSubcoreMesh(core_axis_name="c", subcore_axis_name="s")
    neg = jnp.float32(-jnp.inf)

    def body(feat, srcs, dsts, out, src_b, dst_b, sel_s, sel_d, pend, rows_v,
             acc, sem):
        c = lax.axis_index("c")
        s = lax.axis_index("s")
        wid = s * _NC + c
        lo = wid * rows_per
        lane = lax.iota(jnp.int32, 16)

        def init_acc(i, _):
            for h in range(_D // 16):
                acc[i, pl.ds(16 * h, 16)] = jnp.full((16,), neg, jnp.float32)
            return 0
        lax.fori_loop(0, 256, init_acc, 0, unroll=False)

        def init_sel(i, _):
            sel_s[pl.ds(16 * i, 16)] = jnp.zeros((16,), jnp.int32)
            sel_d[pl.ds(16 * i, 16)] = jnp.zeros((16,), jnp.int32)
            return 0
        lax.fori_loop(0, (_BLK + 16) // 16, init_sel, 0, unroll=False)

        def blk_body(b, _):
            pltpu.sync_copy(srcs.at[pl.ds(b * _BLK, _BLK)], src_b)
            pltpu.sync_copy(dsts.at[pl.ds(b * _BLK, _BLK)], dst_b)

            def cb(v, carry):
                cnt16, npend = carry
                s16 = src_b[pl.ds(16 * v, 16)]
                d16 = dst_b[pl.ds(16 * v, 16)]
                m = (d16 >= lo) & (d16 < lo + rows_per)
                mi = jnp.where(m, jnp.int32(1), jnp.int32(0))
                pref = _prefix_sum16(mi)
                pc = pref[15]
                total = npend + pc

                @pl.when(pc > 0)
                def _append():
                    perm = jnp.zeros((16,), jnp.int32)
                    for l in range(16):
                        tgt = jnp.where(mi[l] == 1, pref[l] - 1, jnp.int32(-1))
                        perm = jnp.where(lane == tgt, jnp.int32(l), perm)
                    cs = _lane_gather(s16, perm)
                    cd = _lane_gather(d16 - lo, perm)
                    ps = pend[pl.ds(0, 16)]
                    pd = pend[pl.ds(16, 16)]
                    sh = jnp.where(lane >= npend, lane - npend, jnp.int32(0))
                    ms = jnp.where(lane < npend, ps, _lane_gather(cs, sh))
                    md = jnp.where(lane < npend, pd, _lane_gather(cd, sh))
                    ovf = jnp.where(lane + 16 - npend < 16, lane + 16 - npend,
                                    jnp.int32(15))
                    os_ = _lane_gather(cs, ovf)
                    od = _lane_gather(cd, ovf)

                    @pl.when(total >= 16)
                    def _emit():
                        sel_s[pl.ds(pl.multiple_of(cnt16, 16), 16)] = ms
                        sel_d[pl.ds(pl.multiple_of(cnt16, 16), 16)] = md
                        pend[pl.ds(0, 16)] = os_
                        pend[pl.ds(16, 16)] = od

                    @pl.when(total < 16)
                    def _hold():
                        pend[pl.ds(0, 16)] = ms
                        pend[pl.ds(16, 16)] = md

                new_cnt16 = jnp.where(total >= 16, cnt16 + 16, cnt16)
                new_np = jnp.where(total >= 16, total - 16, total)
                return new_cnt16, new_np
            cnt16, npend = lax.fori_loop(
                0, _BLK // 16, cb, (jnp.int32(0), jnp.int32(0)), unroll=False)

            @pl.when(npend > 0)
            def _flush():
                sel_s[pl.ds(pl.multiple_of(cnt16, 16), 16)] = pend[pl.ds(0, 16)]
                sel_d[pl.ds(pl.multiple_of(cnt16, 16), 16)] = pend[pl.ds(16, 16)]
            cnt = cnt16 + npend

            def gb(bi, _):
                off = pl.multiple_of(bi * _K, _K)
                pltpu.async_copy(feat.at[sel_s.at[pl.ds(off, _K)]],
                                 rows_v, sem).wait()

                def ab(t, _):
                    eoff = pl.multiple_of(bi * _K + 16 * t, 16)
                    dls = sel_d[pl.ds(eoff, 16)]
                    for l in range(16):
                        e = bi * _K + 16 * t + l
                        dl = dls[l]

                        @pl.when(e < cnt)
                        def _apply():
                            j = 16 * t + l
                            for h in range(_D // 16):
                                sl = pl.ds(16 * h, 16)
                                acc[dl, sl] = jnp.maximum(acc[dl, sl],
                                                          rows_v[j, sl])
                    return 0
                lax.fori_loop(0, _K // 16, ab, 0, unroll=False)
                return 0
            lax.fori_loop(0, lax.div(cnt + (_K - 1), _K), gb, 0, unroll=False)
            return 0
        lax.fori_loop(0, nblk, blk_body, 0, unroll=False)

        def fin(i, _):
            for h in range(_D // 16):
                sl = pl.ds(16 * h, 16)
                v = acc[i, sl]
                acc[i, sl] = jnp.where(v == neg, jnp.float32(0.0), v)
            return 0
        lax.fori_loop(0, 256, fin, 0, unroll=False)
        pltpu.sync_copy(acc, out.at[pl.ds(wid * 256, 256)])

    return pl.kernel(
        body,
        out_type=jax.ShapeDtypeStruct((_NW * 256, _D), jnp.float32),
        mesh=mesh,
        scratch_types=[
            pltpu.VMEM((_BLK,), jnp.int32),
            pltpu.VMEM((_BLK,), jnp.int32),
            pltpu.VMEM((_BLK + 16,), jnp.int32),
            pltpu.VMEM((_BLK + 16,), jnp.int32),
            pltpu.VMEM((32,), jnp.int32),
            pltpu.VMEM((_K, _D), jnp.float32),
            pltpu.VMEM((256, _D), jnp.float32),
            pltpu.SemaphoreType.DMA,
        ],
    )


def _seg_max(feat, src, dst):
    src, dst = _pad_edges(src, dst, _BLK, _NNET)
    rows_per = _NNET // _NW
    out = _make_seg_max(src.shape[0], feat.shape[0])(feat, src, dst)
    return out.reshape(_NW, 256, _D)[:, :rows_per].reshape(_NNET, _D)


def _row_grid(n):
    return _cdiv(n, _BM)


def _bspec(bn=_D):
    return pl.BlockSpec((_BM, bn), lambda i: (i, 0))


def _wspec(di=_D, do=_D):
    return pl.BlockSpec((di, do), lambda i: (0, 0))


def _inv_deg(da, db):
    return 1.0 / jnp.maximum(da[:, 0:1] + db[:, 0:1], 1.0)


def _sage_tc(x, sp, dp, Ws, Wn, b):
    n = x.shape[0]

    def body(x_r, sa_r, sb_r, da_r, db_r, ws_r, wn_r, b_r, o_r):
        nb = (sa_r[:] + sb_r[:]) * _inv_deg(da_r[:], db_r[:])
        o_r[:] = jnp.tanh(x_r[:] @ ws_r[:] + nb @ wn_r[:] + b_r[:])

    return pl.pallas_call(
        body,
        grid=(_row_grid(n),),
        in_specs=[_bspec(), _bspec(), _bspec(), _bspec(16), _bspec(16),
                  _wspec(), _wspec(), pl.BlockSpec((1, _D), lambda i: (0, 0))],
        out_specs=_bspec(),
        out_shape=jax.ShapeDtypeStruct((n, _D), jnp.float32),
    )(x, sp[0], sp[1], dp[0], dp[1], Ws, Wn, b.reshape(1, _D))


def _mean_fin(sp, dp):
    n = sp.shape[1]

    def body(sa_r, sb_r, da_r, db_r, o_r):
        o_r[:] = (sa_r[:] + sb_r[:]) * _inv_deg(da_r[:], db_r[:])

    return pl.pallas_call(
        body,
        grid=(_row_grid(n),),
        in_specs=[_bspec(), _bspec(), _bspec(16), _bspec(16)],
        out_specs=_bspec(),
        out_shape=jax.ShapeDtypeStruct((n, _D), jnp.float32),
    )(sp[0], sp[1], dp[0], dp[1])


def _updense(ap, x, dp, Wsp, Wnp):
    n = x.shape[0]

    def body(aa_r, ab_r, x_r, da_r, db_r, wsa_r, wsb_r, wna_r, wnb_r,
             u_r, g_r):
        a = aa_r[:] + ab_r[:]
        xd = x_r[:] * (da_r[:, 0:1] + db_r[:, 0:1])
        u_r[:] = a @ wsa_r[:] + xd @ wsb_r[:]
        g_r[:] = a @ wna_r[:] + xd @ wnb_r[:]

    return pl.pallas_call(
        body,
        grid=(_row_grid(n),),
        in_specs=[_bspec(), _bspec(), _bspec(), _bspec(16), _bspec(16),
                  _wspec(), _wspec(), _wspec(), _wspec()],
        out_specs=[_bspec(), _bspec()],
        out_shape=[jax.ShapeDtypeStruct((n, _D), jnp.float32),
                   jax.ShapeDtypeStruct((n, _D), jnp.float32)],
    )(ap[0], ap[1], x, dp[0], dp[1],
      Wsp[:_D], Wsp[_D:], Wnp[:_D], Wnp[_D:])


def _addmean_tanh(u, mp, dp, b):
    n = u.shape[0]

    def body(u_r, ma_r, mb_r, da_r, db_r, b_r, o_r):
        m = (ma_r[:] + mb_r[:]) * _inv_deg(da_r[:], db_r[:])
        o_r[:] = jnp.tanh(u_r[:] + m + b_r[:])

    return pl.pallas_call(
        body,
        grid=(_row_grid(n),),
        in_specs=[_bspec(), _bspec(), _bspec(), _bspec(16), _bspec(16),
                  pl.BlockSpec((1, _D), lambda i: (0, 0))],
        out_specs=_bspec(),
        out_shape=jax.ShapeDtypeStruct((n, _D), jnp.float32),
    )(u, mp[0], mp[1], dp[0], dp[1], b.reshape(1, _D))


def _mlp(y, Wm0, bm0, Wm1, bm1):
    n = y.shape[0]
    w1 = jnp.concatenate([Wm1, jnp.zeros((_D, _D - 1), jnp.float32)], axis=1)
    b1 = jnp.concatenate([bm1, jnp.zeros((_D - 1,), jnp.float32)]).reshape(1, _D)

    def body(y_r, w0_r, b0_r, w1_r, b1_r, o_r):
        h = jnp.tanh(y_r[:] @ w0_r[:] + b0_r[:])
        o_r[:] = h @ w1_r[:] + b1_r[:]

    out = pl.pallas_call(
        body,
        grid=(_row_grid(n),),
        in_specs=[_bspec(), _wspec(), pl.BlockSpec((1, _D), lambda i: (0, 0)),
                  _wspec(), pl.BlockSpec((1, _D), lambda i: (0, 0))],
        out_specs=_bspec(),
        out_shape=jax.ShapeDtypeStruct((n, _D), jnp.float32),
    )(y, Wm0, bm0.reshape(1, _D), w1, b1)
    return out[:, 0:1]


def kernel(x, edge_lv0, d01_src, d01_dst, edge_lv1, d12_src, d12_dst, edge_lv2,
           u21_src, u21_dst, u10_src, u10_dst, conn_src, conn_dst,
           Ws0, Wn0, b0, Ws1, Wn1, b1, Ws2, Wn2, b2, Ws3, Wn3, b3,
           Ws4, Wn4, b4, Wm0, bm0, Wm1, bm1):
    lv0s, lv0d = edge_lv0[0], edge_lv0[1]
    lv1s, lv1d = edge_lv1[0], edge_lv1[1]
    lv2s, lv2d = edge_lv2[0], edge_lv2[1]

    dlv0, dlv1, dd01, dd12, dlv2, du21, du10 = _deg_all([
        (lv0d, _N0), (lv1d, _N1), (d01_dst, _N1), (d12_dst, _N2),
        (lv2d, _N2), (u21_dst, _N1), (u10_dst, _N0)])

    s0 = _seg_sum(x, lv0s, lv0d, _N0)
    x0 = _sage_tc(x, s0, dlv0, Ws0, Wn0, b0)
    s1 = _seg_sum(x0, d01_src, d01_dst, _N1)
    p1 = _mean_fin(s1, dd01)
    s2 = _seg_sum(p1, lv1s, lv1d, _N1)
    x1 = _sage_tc(p1, s2, dlv1, Ws1, Wn1, b1)
    s3 = _seg_sum(x1, d12_src, d12_dst, _N2)
    p2 = _mean_fin(s3, dd12)
    s4 = _seg_sum(p2, lv2s, lv2d, _N2)
    x2 = _sage_tc(p2, s4, dlv2, Ws2, Wn2, b2)

    a1 = _seg_sum(x2, u21_src, u21_dst, _N1)
    u1, g1 = _updense(a1, x1, du21, Ws3, Wn3)
    m1 = _seg_sum(g1, lv1s, lv1d, _N1)
    x1u = _addmean_tanh(u1, m1, dlv1, b3)
    a0 = _seg_sum(x1u, u10_src, u10_dst, _N0)
    u0, g0 = _updense(a0, x0, du10, Ws4, Wn4)
    m0 = _seg_sum(g0, lv0s, lv0d, _N0)
    x0u = _addmean_tanh(u0, m0, dlv0, b4)

    y = _seg_max(x0u, conn_src, conn_dst)
    return _mlp(y, Wm0, bm0, Wm1, bm1)
